# baseline (speedup 1.0000x reference)
import dataclasses
import functools

import jax
import jax.numpy as jnp
from jax import lax
from jax.experimental import pallas as pl
from jax.experimental.pallas import tpu as pltpu
from jax.experimental.pallas import tpu_sc as plsc

N = 10000
E = 160000
F = 256
FH = F // 2
DW = 128

NC = 2
NS = 16
HALF = N // NC
EPW = E // NS
K = 128
NB_MAX = (EPW + K) // K + 1
ACC_ROWS = 5120
ZPW = ACC_ROWS // NS
QA = 2504
QB = HALF - QA
ACC_Q = 2560
ZPQ = ACC_Q // NS
KQ = 64
KQSH = 6
CAPR = 160
CAPF = CAPR * KQ

_f32 = jnp.float32
_i32 = jnp.int32


def _compiler_params():
    cp = pltpu.CompilerParams()
    if "needs_layout_passes" in pltpu.CompilerParams.__dataclass_fields__:
        cp = dataclasses.replace(cp, needs_layout_passes=False)
    return cp


def _make_sc_agg():
    mesh = plsc.VectorSubcoreMesh(core_axis_name="c", subcore_axis_name="s")
    out_type = jax.ShapeDtypeStruct((N, F), _f32)

    scratch = [
        pltpu.VMEM((EPW,), _i32),
        pltpu.VMEM((EPW,), _i32),
        pltpu.VMEM((NB_MAX, K), _i32),
        pltpu.VMEM((NB_MAX, K), _i32),
        pltpu.VMEM((NB_MAX, K), _i32),
        pltpu.VMEM((K, FH), _f32),
        pltpu.VMEM((K, FH), _f32),
        pltpu.VMEM((16, FH), _f32),
        pltpu.VMEM_SHARED((ACC_ROWS, FH), _f32),
        pltpu.SMEM((1,), _i32),
        pltpu.SemaphoreType.DMA,
        pltpu.SemaphoreType.DMA,
        pltpu.SemaphoreType.DMA,
    ]

    @functools.partial(pl.kernel, out_type=out_type, mesh=mesh,
                       scratch_types=scratch,
                       compiler_params=_compiler_params())
    def body(x2_hbm, esrc_hbm, edst_hbm, summed_hbm, src_stage, dst_stage,
             sel0, sel1, dst_sel, rows0, rows1, zbuf, acc, cnt_ref,
             gsem, ssem0, ssem1):
        c_idx = lax.axis_index("c")
        s_idx = lax.axis_index("s")
        iota16 = lax.iota(_i32, 16)
        zero16 = jnp.zeros((16,), _f32)

        @pl.loop(0, 16)
        def _(r):
            @pl.loop(0, FH, step=16)
            def _(k):
                zbuf[r, pl.ds(k, 16)] = zero16

        lo = s_idx * EPW
        pltpu.sync_copy(esrc_hbm.at[pl.ds(lo, EPW)], src_stage)
        pltpu.sync_copy(edst_hbm.at[pl.ds(lo, EPW)], dst_stage)

        base = c_idx * HALF
        cnt_ref[0] = 0

        @pl.loop(0, EPW, step=16)
        def _(i):
            sv = src_stage[pl.ds(i, 16)]
            dv = dst_stage[pl.ds(i, 16)]
            dl = dv - base
            mask = (dl >= 0) & (dl < HALF)
            mi = mask.astype(_i32)
            cnt = cnt_ref[0]
            pos = cnt + plsc.cumsum(mi) - 1
            row = lax.shift_right_logical(pos, 7)
            col = pos & (K - 1)
            sv2 = sv + sv
            plsc.store_scatter(sel0, [row, col], sv2, mask=mask)
            plsc.store_scatter(sel1, [row, col], sv2 + 1, mask=mask)
            plsc.store_scatter(dst_sel, [row, col], dl, mask=mask)
            cnt_ref[0] = cnt + jnp.sum(mi)

        cnt = cnt_ref[0]
        pad_dst = iota16 + HALF
        for j in range(K // 16):
            p = cnt + iota16 + (j * 16)
            row = lax.shift_right_logical(p, 7)
            col = p & (K - 1)
            pad_src = (iota16 + (j * 16)) * 2
            plsc.store_scatter(sel0, [row, col], pad_src)
            plsc.store_scatter(sel1, [row, col], pad_src + 1)
            plsc.store_scatter(dst_sel, [row, col], pad_dst)

        nb = lax.shift_right_logical(cnt + (K - 1), 7)
        zoff = s_idx * ZPW
        obase = c_idx * HALF
        off = s_idx * 312

        for half in range(2):
            sel = sel0 if half == 0 else sel1

            @pl.loop(0, ZPW, step=16)
            def _(t):
                pltpu.sync_copy(zbuf, acc.at[pl.ds(zoff + t, 16)])

            plsc.subcore_barrier()

            def batch_body(b, carry):
                def do(rows, ssem):
                    @pl.when((b >= 2) & (b - 2 < nb))
                    def _():
                        pltpu.make_async_copy(
                            rows, acc.at[dst_sel.at[b]], ssem).wait()

                    @pl.when(b < nb)
                    def _():
                        pltpu.async_copy(x2_hbm.at[sel.at[b]], rows,
                                         gsem).wait()
                        pltpu.async_copy(rows, acc.at[dst_sel.at[b]], ssem,
                                         add=True)

                @pl.when((b & 1) == 0)
                def _():
                    do(rows0, ssem0)

                @pl.when((b & 1) == 1)
                def _():
                    do(rows1, ssem1)

                return carry

            lax.fori_loop(0, nb + 2, batch_body, jnp.int32(0))

            plsc.subcore_barrier()

            hoff = half * FH
            pltpu.sync_copy(
                acc.at[pl.ds(off, 312)],
                summed_hbm.at[pl.ds(obase + off, 312), pl.ds(hoff, FH)])

            @pl.when(s_idx == 0)
            def _():
                pltpu.sync_copy(
                    acc.at[pl.ds(4992, 8)],
                    summed_hbm.at[pl.ds(obase + 4992, 8), pl.ds(hoff, FH)])

            if half == 0:
                plsc.subcore_barrier()

    return body


def _make_sc_deg():
    mesh = plsc.VectorSubcoreMesh(core_axis_name="c", subcore_axis_name="s")
    out_type = jax.ShapeDtypeStruct((N, DW), _f32)

    scratch = [
        pltpu.VMEM((EPW,), _i32),
        pltpu.VMEM((NB_MAX, K), _i32),
        pltpu.VMEM((K, DW), _f32),
        pltpu.VMEM((16, DW), _f32),
        pltpu.VMEM_SHARED((ACC_ROWS, DW), _f32),
        pltpu.SMEM((1,), _i32),
        pltpu.SemaphoreType.DMA,
    ]

    @functools.partial(pl.kernel, out_type=out_type, mesh=mesh,
                       scratch_types=scratch,
                       compiler_params=_compiler_params())
    def body(edst_hbm, deg_hbm, dst_stage, dst_sel, ones_rows, zbuf16,
             degacc, cnt_ref, ssem):
        c_idx = lax.axis_index("c")
        s_idx = lax.axis_index("s")
        iota16 = lax.iota(_i32, 16)
        zero16 = jnp.zeros((16,), _f32)
        one16 = jnp.ones((16,), _f32)

        @pl.loop(0, 16)
        def _(r):
            @pl.loop(0, DW, step=16)
            def _(k):
                zbuf16[r, pl.ds(k, 16)] = zero16

        @pl.loop(0, K)
        def _(r):
            @pl.loop(0, DW, step=16)
            def _(k):
                ones_rows[r, pl.ds(k, 16)] = one16

        zoff = s_idx * ZPW

        @pl.loop(0, ZPW, step=16)
        def _(t):
            pltpu.sync_copy(zbuf16, degacc.at[pl.ds(zoff + t, 16)])

        lo = s_idx * EPW
        pltpu.sync_copy(edst_hbm.at[pl.ds(lo, EPW)], dst_stage)

        base = c_idx * HALF
        cnt_ref[0] = 0

        @pl.loop(0, EPW, step=16)
        def _(i):
            dv = dst_stage[pl.ds(i, 16)]
            dl = dv - base
            mask = (dl >= 0) & (dl < HALF)
            mi = mask.astype(_i32)
            cnt = cnt_ref[0]
            pos = cnt + plsc.cumsum(mi) - 1
            row = lax.shift_right_logical(pos, 7)
            col = pos & (K - 1)
            plsc.store_scatter(dst_sel, [row, col], dl, mask=mask)
            cnt_ref[0] = cnt + jnp.sum(mi)

        cnt = cnt_ref[0]
        pad_dst = iota16 + HALF
        for j in range(K // 16):
            p = cnt + iota16 + (j * 16)
            row = lax.shift_right_logical(p, 7)
            col = p & (K - 1)
            plsc.store_scatter(dst_sel, [row, col], pad_dst)

        nb = lax.shift_right_logical(cnt + (K - 1), 7)

        plsc.subcore_barrier()

        def batch_body(b, carry):
            pltpu.async_copy(ones_rows, degacc.at[dst_sel.at[b]], ssem,
                             add=True)
            return carry

        lax.fori_loop(0, nb, batch_body, jnp.int32(0))

        def drain_body(b, carry):
            pltpu.make_async_copy(ones_rows, degacc.at[dst_sel.at[0]],
                                  ssem).wait()
            return carry

        lax.fori_loop(0, nb, drain_body, jnp.int32(0))

        plsc.subcore_barrier()

        obase = c_idx * HALF
        off = s_idx * 312
        pltpu.sync_copy(degacc.at[pl.ds(off, 312)],
                        deg_hbm.at[pl.ds(obase + off, 312)])

        @pl.when(s_idx == 0)
        def _():
            pltpu.sync_copy(degacc.at[pl.ds(4992, 8)],
                            deg_hbm.at[pl.ds(obase + 4992, 8)])

    return body


_sc_agg_kernel = _make_sc_agg()
_sc_deg_kernel = _make_sc_deg()


def _tc_body(x_ref, s_ref, d_ref, w_ref, o_ref):
    deg = jnp.maximum(d_ref[:, 0:1], 1.0)
    hn = s_ref[...] / deg
    h = jnp.dot(x_ref[...], w_ref[0:F, :], preferred_element_type=_f32)
    h = h + jnp.dot(hn, w_ref[F:2 * F, :], preferred_element_type=_f32)
    nrm = jnp.sqrt(jnp.sum(h * h, axis=1, keepdims=True))
    o_ref[...] = h / (nrm + 1e-4)


def _tc_layer(xin, summed, deg, w):
    bm = 1000
    return pl.pallas_call(
        _tc_body,
        grid=(N // bm,),
        in_specs=[
            pl.BlockSpec((bm, F), lambda i: (i, 0)),
            pl.BlockSpec((bm, F), lambda i: (i, 0)),
            pl.BlockSpec((bm, DW), lambda i: (i, 0)),
            pl.BlockSpec((2 * F, F), lambda i: (0, 0)),
        ],
        out_specs=pl.BlockSpec((bm, F), lambda i: (i, 0)),
        out_shape=jax.ShapeDtypeStruct((N, F), _f32),
    )(xin, summed, deg, w)


@jax.jit
def kernel(x, edge_index, weight_in, weight_out):
    esrc = edge_index[0]
    edst = edge_index[1]
    deg = _sc_deg_kernel(edst)
    x2 = x.reshape(2 * N, FH)
    summed1 = _sc_agg_kernel(x2, esrc, edst)
    h = _tc_layer(x, summed1, deg, weight_in)
    summed2 = _sc_agg_kernel(h.reshape(2 * N, FH), esrc, edst)
    return _tc_layer(h, summed2, deg, weight_out)


# speedup vs baseline: 1.0597x; 1.0597x over previous
import dataclasses
import functools

import jax
import jax.numpy as jnp
from jax import lax
from jax.experimental import pallas as pl
from jax.experimental.pallas import tpu as pltpu
from jax.experimental.pallas import tpu_sc as plsc

N = 10000
E = 160000
F = 256
FH = F // 2
DW = 128

NC = 2
NS = 16
HALF = N // NC
EPW = E // NS
K = 128
NB_MAX = (EPW + K) // K + 1
ACC_ROWS = 5120
ZPW = ACC_ROWS // NS
QA = 2504
QB = HALF - QA
ACC_Q = 2560
ZPQ = ACC_Q // NS
KQ = 64
KQSH = 6
CAPR = 160
CAPF = CAPR * KQ

_f32 = jnp.float32
_i32 = jnp.int32


def _vgather(vec, idx):
    dn = lax.GatherDimensionNumbers(offset_dims=(), collapsed_slice_dims=(0,),
                                    start_index_map=(0,))
    return lax.gather(vec, idx[:, None], dn, slice_sizes=(1,),
                      mode=lax.GatherScatterMode.PROMISE_IN_BOUNDS)


def _compiler_params():
    cp = pltpu.CompilerParams()
    if "needs_layout_passes" in pltpu.CompilerParams.__dataclass_fields__:
        cp = dataclasses.replace(cp, needs_layout_passes=False)
    return cp


def _make_sc_agg():
    mesh = plsc.VectorSubcoreMesh(core_axis_name="c", subcore_axis_name="s")
    out_type = jax.ShapeDtypeStruct((N, F), _f32)

    scratch = [
        pltpu.VMEM((EPW,), _i32),
        pltpu.VMEM((EPW,), _i32),
        pltpu.VMEM((NB_MAX, K), _i32),
        pltpu.VMEM((NB_MAX, K), _i32),
        pltpu.VMEM((NB_MAX, K), _i32),
        pltpu.VMEM((K, FH), _f32),
        pltpu.VMEM((K, FH), _f32),
        pltpu.VMEM((16, FH), _f32),
        pltpu.VMEM_SHARED((ACC_ROWS, FH), _f32),
        pltpu.SMEM((1,), _i32),
        pltpu.SemaphoreType.DMA,
        pltpu.SemaphoreType.DMA,
        pltpu.SemaphoreType.DMA,
    ]

    @functools.partial(pl.kernel, out_type=out_type, mesh=mesh,
                       scratch_types=scratch,
                       compiler_params=_compiler_params())
    def body(x2_hbm, esrc_hbm, edst_hbm, summed_hbm, src_stage, dst_stage,
             sel0, sel1, dst_sel, rows0, rows1, zbuf, acc, cnt_ref,
             gsem, ssem0, ssem1):
        c_idx = lax.axis_index("c")
        s_idx = lax.axis_index("s")
        iota16 = lax.iota(_i32, 16)
        zero16 = jnp.zeros((16,), _f32)

        @pl.loop(0, 16)
        def _(r):
            @pl.loop(0, FH, step=16)
            def _(k):
                zbuf[r, pl.ds(k, 16)] = zero16

        lo = s_idx * EPW
        pltpu.sync_copy(esrc_hbm.at[pl.ds(lo, EPW)], src_stage)
        pltpu.sync_copy(edst_hbm.at[pl.ds(lo, EPW)], dst_stage)

        base = c_idx * HALF
        cnt_ref[0] = 0

        @pl.loop(0, EPW, step=16)
        def _(i):
            sv = src_stage[pl.ds(i, 16)]
            dv = dst_stage[pl.ds(i, 16)]
            dl = dv - base
            mask = (dl >= 0) & (dl < HALF)
            mi = mask.astype(_i32)
            cnt = cnt_ref[0]
            pos = cnt + plsc.cumsum(mi) - 1
            row = lax.shift_right_logical(pos, 7)
            col = pos & (K - 1)
            sv2 = sv + sv
            plsc.store_scatter(sel0, [row, col], sv2, mask=mask)
            plsc.store_scatter(sel1, [row, col], sv2 + 1, mask=mask)
            plsc.store_scatter(dst_sel, [row, col], dl, mask=mask)
            cnt_ref[0] = cnt + jnp.sum(mi)

        cnt = cnt_ref[0]
        pad_dst = iota16 + HALF
        for j in range(K // 16):
            p = cnt + iota16 + (j * 16)
            row = lax.shift_right_logical(p, 7)
            col = p & (K - 1)
            pad_src = (iota16 + (j * 16)) * 2
            plsc.store_scatter(sel0, [row, col], pad_src)
            plsc.store_scatter(sel1, [row, col], pad_src + 1)
            plsc.store_scatter(dst_sel, [row, col], pad_dst)

        nb = lax.shift_right_logical(cnt + (K - 1), 7)
        zoff = s_idx * ZPW
        obase = c_idx * HALF
        off = s_idx * 312

        for half in range(2):
            sel = sel0 if half == 0 else sel1

            @pl.loop(0, ZPW, step=16)
            def _(t):
                pltpu.sync_copy(zbuf, acc.at[pl.ds(zoff + t, 16)])

            plsc.subcore_barrier()

            def batch_body(b, carry):
                def do(rows, ssem):
                    @pl.when((b >= 2) & (b - 2 < nb))
                    def _():
                        pltpu.make_async_copy(
                            rows, acc.at[dst_sel.at[b]], ssem).wait()

                    @pl.when(b < nb)
                    def _():
                        pltpu.async_copy(x2_hbm.at[sel.at[b]], rows,
                                         gsem).wait()
                        pltpu.async_copy(rows, acc.at[dst_sel.at[b]], ssem,
                                         add=True)

                @pl.when((b & 1) == 0)
                def _():
                    do(rows0, ssem0)

                @pl.when((b & 1) == 1)
                def _():
                    do(rows1, ssem1)

                return carry

            lax.fori_loop(0, nb + 2, batch_body, jnp.int32(0))

            plsc.subcore_barrier()

            hoff = half * FH
            pltpu.sync_copy(
                acc.at[pl.ds(off, 312)],
                summed_hbm.at[pl.ds(obase + off, 312), pl.ds(hoff, FH)])

            @pl.when(s_idx == 0)
            def _():
                pltpu.sync_copy(
                    acc.at[pl.ds(4992, 8)],
                    summed_hbm.at[pl.ds(obase + 4992, 8), pl.ds(hoff, FH)])

            if half == 0:
                plsc.subcore_barrier()

    return body


def _make_sc_deg():
    mesh = plsc.VectorSubcoreMesh(core_axis_name="c", subcore_axis_name="s")
    out_type = jax.ShapeDtypeStruct((NC * 40, K), _f32)

    scratch = [
        pltpu.VMEM((EPW,), _i32),
        pltpu.VMEM((40, K), _f32),
        pltpu.VMEM((16, K), _f32),
        pltpu.VMEM((40,), _i32),
        pltpu.VMEM_SHARED((40, K), _f32),
    ]

    @functools.partial(pl.kernel, out_type=out_type, mesh=mesh,
                       scratch_types=scratch,
                       compiler_params=_compiler_params())
    def body(edst_hbm, deg_hbm, dst_stage, hist, zbuf16, idx40, degacc):
        c_idx = lax.axis_index("c")
        s_idx = lax.axis_index("s")
        iota16 = lax.iota(_i32, 16)
        zero16 = jnp.zeros((16,), _f32)

        @pl.loop(0, 16)
        def _(r):
            @pl.loop(0, K, step=16)
            def _(k):
                zbuf16[r, pl.ds(k, 16)] = zero16

        @pl.loop(0, 40)
        def _(r):
            @pl.loop(0, K, step=16)
            def _(k):
                hist[r, pl.ds(k, 16)] = zero16

        idx40[pl.ds(0, 16)] = iota16
        idx40[pl.ds(16, 16)] = iota16 + 16
        idx40[pl.ds(24, 16)] = iota16 + 24

        @pl.when(s_idx == 0)
        def _():
            pltpu.sync_copy(zbuf16, degacc.at[pl.ds(0, 16)])
            pltpu.sync_copy(zbuf16, degacc.at[pl.ds(16, 16)])
            pltpu.sync_copy(zbuf16.at[pl.ds(0, 8)], degacc.at[pl.ds(32, 8)])

        lo = s_idx * EPW
        pltpu.sync_copy(edst_hbm.at[pl.ds(lo, EPW)], dst_stage)

        base = c_idx * HALF

        @pl.loop(0, EPW, step=16)
        def _(i):
            dv = dst_stage[pl.ds(i, 16)]
            dl = dv - base
            mask = (dl >= 0) & (dl < HALF)
            dm = jnp.where(mask, dl, HALF + iota16)
            s, _ = plsc.sort_key_val(dm, dm)
            prev = _vgather(s, jnp.maximum(iota16 - 1, 0))
            is_head = (iota16 == 0) | (s != prev)
            a = jnp.where(is_head, iota16, 16)
            nxt = _vgather(a, jnp.minimum(iota16 + 1, 15))
            a2 = jnp.where(iota16 == 15, 16, nxt)
            m = 31 - a2
            nh = 31 - lax.rev(plsc.cummax(lax.rev(m, (0,))), (0,))
            counts = (nh - iota16).astype(_f32)
            plsc.addupdate_scatter(
                hist, [lax.shift_right_logical(s, 7), s & (K - 1)],
                counts, mask=is_head)

        plsc.subcore_barrier()

        pltpu.sync_copy(hist, degacc.at[idx40], add=True)

        plsc.subcore_barrier()

        @pl.when(s_idx == 0)
        def _():
            pltpu.sync_copy(degacc, deg_hbm.at[pl.ds(c_idx * 40, 40)])

    return body


_sc_agg_kernel = _make_sc_agg()
_sc_deg_kernel = _make_sc_deg()


def _tc_body(x_ref, s_ref, d_ref, w_ref, o_ref):
    deg = jnp.maximum(d_ref[:, 0:1], 1.0)
    hn = s_ref[...] / deg
    h = jnp.dot(x_ref[...], w_ref[0:F, :], preferred_element_type=_f32)
    h = h + jnp.dot(hn, w_ref[F:2 * F, :], preferred_element_type=_f32)
    nrm = jnp.sqrt(jnp.sum(h * h, axis=1, keepdims=True))
    o_ref[...] = h / (nrm + 1e-4)


def _tc_layer(xin, summed, deg, w):
    bm = 1000
    return pl.pallas_call(
        _tc_body,
        grid=(N // bm,),
        in_specs=[
            pl.BlockSpec((bm, F), lambda i: (i, 0)),
            pl.BlockSpec((bm, F), lambda i: (i, 0)),
            pl.BlockSpec((bm, 1), lambda i: (i, 0)),
            pl.BlockSpec((2 * F, F), lambda i: (0, 0)),
        ],
        out_specs=pl.BlockSpec((bm, F), lambda i: (i, 0)),
        out_shape=jax.ShapeDtypeStruct((N, F), _f32),
    )(xin, summed, deg, w)


@jax.jit
def kernel(x, edge_index, weight_in, weight_out):
    esrc = edge_index[0]
    edst = edge_index[1]
    deg_packed = _sc_deg_kernel(edst)
    deg = deg_packed.reshape(NC, 40 * K)[:, :HALF].reshape(N, 1)
    x2 = x.reshape(2 * N, FH)
    summed1 = _sc_agg_kernel(x2, esrc, edst)
    h = _tc_layer(x, summed1, deg, weight_in)
    summed2 = _sc_agg_kernel(h.reshape(2 * N, FH), esrc, edst)
    return _tc_layer(h, summed2, deg, weight_out)
